# baseline (speedup 1.0000x reference)
import functools

import jax
import jax.numpy as jnp
from jax import lax
from jax.experimental import pallas as pl
from jax.experimental.pallas import tpu as pltpu
from jax.experimental.pallas import tpu_sc as plsc

_VOCAB = 1_000_000
_EMB = 64
_NROWG = 8
_NCOLG = 4
_COLG = 249_984
_CHUNK = 3_968
_NCH = _COLG // _CHUNK
_TAIL_BASE = _NCOLG * _COLG
_TAIL = _VOCAB - _TAIL_BASE


def _sc_body(in_hbm, out_hbm, buf, tail_buf, in_sem0, in_sem1, out_sem0, out_sem1):
    in_sems = (in_sem0, in_sem1)
    out_sems = (out_sem0, out_sem1)
    wid = lax.axis_index("s") * 2 + lax.axis_index("c")
    rowg = wid // _NCOLG
    colg = wid % _NCOLG
    row0 = pl.multiple_of(rowg * _NROWG, 8)
    col_base = colg * _COLG

    def cols(i):
        return pl.ds(pl.multiple_of(col_base + i * _CHUNK, 128), _CHUNK)

    def in_copy(i, slot):
        return pltpu.make_async_copy(
            in_hbm.at[pl.ds(row0, _NROWG), cols(i)],
            buf.at[slot],
            in_sems[slot],
        )

    def out_copy(i, slot):
        return pltpu.make_async_copy(
            buf.at[slot],
            out_hbm.at[pl.ds(row0, _NROWG), cols(i)],
            out_sems[slot],
        )

    in_copy(0, 0).start()
    in_copy(0, 0).wait()
    out_copy(0, 0).start()
    in_copy(1, 1).start()
    in_copy(1, 1).wait()
    out_copy(1, 1).start()
    out_copy(0, 0).wait()
    in_copy(2, 0).start()

    def pair(g, _):
        for b in (0, 1):
            i = 2 * g + b
            in_copy(i, b).wait()
            out_copy(i, b).start()
            out_copy(i - 1, 1 - b).wait()
            in_copy(i + 1, 1 - b).start()
        return ()

    lax.fori_loop(1, (_NCH - 3) // 2, pair, (), unroll=1)

    i = _NCH - 3
    in_copy(i, 0).wait()
    out_copy(i, 0).start()
    out_copy(i - 1, 1).wait()
    in_copy(i + 1, 1).start()
    in_copy(i + 1, 1).wait()
    out_copy(i + 1, 1).start()
    out_copy(i, 0).wait()
    in_copy(i + 2, 0).start()
    in_copy(i + 2, 0).wait()
    out_copy(i + 2, 0).start()
    out_copy(i + 1, 1).wait()
    out_copy(i + 2, 0).wait()

    @pl.when(wid == _NROWG * _NCOLG - 1)
    def _():
        t_in = pltpu.make_async_copy(
            in_hbm.at[:, pl.ds(_TAIL_BASE, _TAIL)], tail_buf, in_sems[0]
        )
        t_out = pltpu.make_async_copy(
            tail_buf, out_hbm.at[:, pl.ds(_TAIL_BASE, _TAIL)], out_sems[0]
        )
        t_in.start()
        t_in.wait()
        t_out.start()
        t_out.wait()


def _sc_copy(W_t):
    mesh = plsc.VectorSubcoreMesh(core_axis_name="c", subcore_axis_name="s")
    k = functools.partial(
        pl.kernel,
        mesh=mesh,
        out_type=jax.ShapeDtypeStruct((_EMB, _VOCAB), jnp.float32),
        scratch_types=[
            pltpu.VMEM((2, _NROWG, _CHUNK), jnp.float32),
            pltpu.VMEM((_EMB, _TAIL), jnp.float32),
            pltpu.SemaphoreType.DMA,
            pltpu.SemaphoreType.DMA,
            pltpu.SemaphoreType.DMA,
            pltpu.SemaphoreType.DMA,
        ],
    )(_sc_body)
    return k(W_t)


def kernel(lang, W_emb):
    del lang
    return _sc_copy(W_emb.T).T


# speedup vs baseline: 1.0081x; 1.0081x over previous
import functools

import jax
import jax.numpy as jnp
from jax import lax
from jax.experimental import pallas as pl
from jax.experimental.pallas import tpu as pltpu
from jax.experimental.pallas import tpu_sc as plsc

_VOCAB = 1_000_000
_EMB = 64
_NROWG = 8
_NCOLG = 4
_COLG = 249_984
_CHUNK = 3_968
_NCH = _COLG // _CHUNK
_NSLOT = 4
_TAIL_BASE = _NCOLG * _COLG
_TAIL = _VOCAB - _TAIL_BASE


def _sc_body(in_hbm, out_hbm, buf, tail_buf, *sems):
    in_sems = sems[:_NSLOT]
    out_sems = sems[_NSLOT:]
    wid = lax.axis_index("s") * 2 + lax.axis_index("c")
    rowg = wid // _NCOLG
    colg = wid % _NCOLG
    row0 = pl.multiple_of(rowg * _NROWG, 8)
    col_base = colg * _COLG

    def cols(i):
        return pl.ds(pl.multiple_of(col_base + i * _CHUNK, 128), _CHUNK)

    def in_copy(i, slot):
        return pltpu.make_async_copy(
            in_hbm.at[pl.ds(row0, _NROWG), cols(i)],
            buf.at[slot],
            in_sems[slot],
        )

    def out_copy(i, slot):
        return pltpu.make_async_copy(
            buf.at[slot],
            out_hbm.at[pl.ds(row0, _NROWG), cols(i)],
            out_sems[slot],
        )

    in_copy(0, 0).start()
    in_copy(1, 1).start()
    in_copy(0, 0).wait()
    out_copy(0, 0).start()
    in_copy(2, 2).start()
    in_copy(1, 1).wait()
    out_copy(1, 1).start()
    in_copy(3, 3).start()
    in_copy(2, 2).wait()
    out_copy(2, 2).start()
    out_copy(0, 0).wait()
    in_copy(4, 0).start()
    in_copy(3, 3).wait()
    out_copy(3, 3).start()
    out_copy(1, 1).wait()
    in_copy(5, 1).start()

    def quad(g, _):
        for b in range(4):
            j = 4 * g + b
            s2 = (b + 2) % 4
            in_copy(j, b).wait()
            out_copy(j, b).start()
            out_copy(j - 2, s2).wait()
            in_copy(j + 2, s2).start()
        return ()

    lax.fori_loop(1, (_NCH - 3) // 4, quad, (), unroll=1)

    j = _NCH - 3
    in_copy(j, 0).wait()
    out_copy(j, 0).start()
    out_copy(j - 2, 2).wait()
    in_copy(j + 2, 2).start()
    in_copy(j + 1, 1).wait()
    out_copy(j + 1, 1).start()
    out_copy(j - 1, 3).wait()
    in_copy(j + 2, 2).wait()
    out_copy(j + 2, 2).start()
    out_copy(j, 0).wait()
    out_copy(j + 1, 1).wait()
    out_copy(j + 2, 2).wait()

    @pl.when(colg == _NCOLG - 1)
    def _():
        t_in = pltpu.make_async_copy(
            in_hbm.at[pl.ds(row0, _NROWG), pl.ds(_TAIL_BASE, _TAIL)],
            tail_buf,
            in_sems[0],
        )
        t_out = pltpu.make_async_copy(
            tail_buf,
            out_hbm.at[pl.ds(row0, _NROWG), pl.ds(_TAIL_BASE, _TAIL)],
            out_sems[0],
        )
        t_in.start()
        t_in.wait()
        t_out.start()
        t_out.wait()


def _sc_copy(W_t):
    mesh = plsc.VectorSubcoreMesh(core_axis_name="c", subcore_axis_name="s")
    k = functools.partial(
        pl.kernel,
        mesh=mesh,
        out_type=jax.ShapeDtypeStruct((_EMB, _VOCAB), jnp.float32),
        scratch_types=[
            pltpu.VMEM((_NSLOT, _NROWG, _CHUNK), jnp.float32),
            pltpu.VMEM((_NROWG, _TAIL), jnp.float32),
        ]
        + [pltpu.SemaphoreType.DMA] * (2 * _NSLOT),
    )(_sc_body)
    return k(W_t)


def kernel(lang, W_emb):
    del lang
    return _sc_copy(W_emb.T).T


# speedup vs baseline: 1.2367x; 1.2268x over previous
import jax
import jax.numpy as jnp
from jax.experimental import pallas as pl
from jax.experimental.pallas import tpu as pltpu

_VOCAB = 1_000_000
_EMB = 64
_CHUNK = 7_936
_NCH = 126
_TAIL_BASE = _NCH * _CHUNK
_TAIL = _VOCAB - _TAIL_BASE
_K = 8
_L = 4


def _tc_body(in_hbm, out_hbm, buf, tail_buf, in_sems, out_sems):
    def in_copy(j):
        s = j % _K
        return pltpu.make_async_copy(
            in_hbm.at[:, pl.ds(j * _CHUNK, _CHUNK)], buf.at[s], in_sems.at[s]
        )

    def out_copy(j):
        s = j % _K
        return pltpu.make_async_copy(
            buf.at[s], out_hbm.at[:, pl.ds(j * _CHUNK, _CHUNK)], out_sems.at[s]
        )

    waited = set()
    for j in range(_L):
        in_copy(j).start()
    for j in range(_NCH):
        in_copy(j).wait()
        out_copy(j).start()
        nxt = j + _L
        if nxt < _NCH:
            prev = nxt - _K
            if prev >= 0:
                out_copy(prev).wait()
                waited.add(prev)
            in_copy(nxt).start()
    for j in range(_NCH):
        if j not in waited:
            out_copy(j).wait()

    t_in = pltpu.make_async_copy(
        in_hbm.at[:, pl.ds(_TAIL_BASE, _TAIL)], tail_buf, in_sems.at[0]
    )
    t_out = pltpu.make_async_copy(
        tail_buf, out_hbm.at[:, pl.ds(_TAIL_BASE, _TAIL)], out_sems.at[0]
    )
    t_in.start()
    t_in.wait()
    t_out.start()
    t_out.wait()


def kernel(lang, W_emb):
    del lang
    W_t = W_emb.T
    out = pl.pallas_call(
        _tc_body,
        in_specs=[pl.BlockSpec(memory_space=pltpu.MemorySpace.HBM)],
        out_specs=pl.BlockSpec(memory_space=pltpu.MemorySpace.HBM),
        out_shape=jax.ShapeDtypeStruct((_EMB, _VOCAB), jnp.float32),
        scratch_shapes=[
            pltpu.VMEM((_K, _EMB, _CHUNK), jnp.float32),
            pltpu.VMEM((_EMB, _TAIL), jnp.float32),
            pltpu.SemaphoreType.DMA((_K,)),
            pltpu.SemaphoreType.DMA((_K,)),
        ],
    )(W_t)
    return out.T


# speedup vs baseline: 1.2427x; 1.0049x over previous
import jax
import jax.numpy as jnp
from jax.experimental import pallas as pl
from jax.experimental.pallas import tpu as pltpu

_VOCAB = 1_000_000
_EMB = 64
_CHUNK = 7_936
_NCH = 126
_TAIL_BASE = _NCH * _CHUNK
_TAIL = _VOCAB - _TAIL_BASE
_K = 12
_L = 6


def _tc_body(in_hbm, out_hbm, buf, tail_buf, in_sems, out_sems, tail_sems):
    def in_copy(j):
        s = j % _K
        return pltpu.make_async_copy(
            in_hbm.at[:, pl.ds(j * _CHUNK, _CHUNK)], buf.at[s], in_sems.at[s]
        )

    def out_copy(j):
        s = j % _K
        return pltpu.make_async_copy(
            buf.at[s], out_hbm.at[:, pl.ds(j * _CHUNK, _CHUNK)], out_sems.at[s]
        )

    t_in = pltpu.make_async_copy(
        in_hbm.at[:, pl.ds(_TAIL_BASE, _TAIL)], tail_buf, tail_sems.at[0]
    )
    t_out = pltpu.make_async_copy(
        tail_buf, out_hbm.at[:, pl.ds(_TAIL_BASE, _TAIL)], tail_sems.at[1]
    )
    t_in.start()

    waited = set()
    for j in range(_L):
        in_copy(j).start()
    t_in.wait()
    t_out.start()
    for j in range(_NCH):
        in_copy(j).wait()
        out_copy(j).start()
        nxt = j + _L
        if nxt < _NCH:
            prev = nxt - _K
            if prev >= 0:
                out_copy(prev).wait()
                waited.add(prev)
            in_copy(nxt).start()
    for j in range(_NCH):
        if j not in waited:
            out_copy(j).wait()
    t_out.wait()


def kernel(lang, W_emb):
    del lang
    W_t = W_emb.T
    out = pl.pallas_call(
        _tc_body,
        in_specs=[pl.BlockSpec(memory_space=pltpu.MemorySpace.HBM)],
        out_specs=pl.BlockSpec(memory_space=pltpu.MemorySpace.HBM),
        out_shape=jax.ShapeDtypeStruct((_EMB, _VOCAB), jnp.float32),
        scratch_shapes=[
            pltpu.VMEM((_K, _EMB, _CHUNK), jnp.float32),
            pltpu.VMEM((_EMB, _TAIL), jnp.float32),
            pltpu.SemaphoreType.DMA((_K,)),
            pltpu.SemaphoreType.DMA((_K,)),
            pltpu.SemaphoreType.DMA((2,)),
        ],
    )(W_t)
    return out.T


# speedup vs baseline: 1.2472x; 1.0036x over previous
import jax
import jax.numpy as jnp
from jax.experimental import pallas as pl
from jax.experimental.pallas import tpu as pltpu

_VOCAB = 1_000_000
_EMB = 64
_CHUNK = 15_872
_NCH = 63
_TAIL_BASE = _NCH * _CHUNK
_TAIL = _VOCAB - _TAIL_BASE
_K = 6
_L = 3


def _tc_body(in_hbm, out_hbm, buf, tail_buf, in_sems, out_sems, tail_sems):
    def in_copy(j):
        s = j % _K
        return pltpu.make_async_copy(
            in_hbm.at[:, pl.ds(j * _CHUNK, _CHUNK)], buf.at[s], in_sems.at[s]
        )

    def out_copy(j):
        s = j % _K
        return pltpu.make_async_copy(
            buf.at[s], out_hbm.at[:, pl.ds(j * _CHUNK, _CHUNK)], out_sems.at[s]
        )

    t_in = pltpu.make_async_copy(
        in_hbm.at[:, pl.ds(_TAIL_BASE, _TAIL)], tail_buf, tail_sems.at[0]
    )
    t_out = pltpu.make_async_copy(
        tail_buf, out_hbm.at[:, pl.ds(_TAIL_BASE, _TAIL)], tail_sems.at[1]
    )
    t_in.start()

    waited = set()
    for j in range(_L):
        in_copy(j).start()
    t_in.wait()
    t_out.start()
    for j in range(_NCH):
        in_copy(j).wait()
        out_copy(j).start()
        nxt = j + _L
        if nxt < _NCH:
            prev = nxt - _K
            if prev >= 0:
                out_copy(prev).wait()
                waited.add(prev)
            in_copy(nxt).start()
    for j in range(_NCH):
        if j not in waited:
            out_copy(j).wait()
    t_out.wait()


def kernel(lang, W_emb):
    del lang
    W_t = W_emb.T
    out = pl.pallas_call(
        _tc_body,
        in_specs=[pl.BlockSpec(memory_space=pltpu.MemorySpace.HBM)],
        out_specs=pl.BlockSpec(memory_space=pltpu.MemorySpace.HBM),
        out_shape=jax.ShapeDtypeStruct((_EMB, _VOCAB), jnp.float32),
        scratch_shapes=[
            pltpu.VMEM((_K, _EMB, _CHUNK), jnp.float32),
            pltpu.VMEM((_EMB, _TAIL), jnp.float32),
            pltpu.SemaphoreType.DMA((_K,)),
            pltpu.SemaphoreType.DMA((_K,)),
            pltpu.SemaphoreType.DMA((2,)),
        ],
    )(W_t)
    return out.T


# speedup vs baseline: 1.2505x; 1.0026x over previous
import jax
import jax.numpy as jnp
from jax.experimental import pallas as pl
from jax.experimental.pallas import tpu as pltpu

_VOCAB = 1_000_000
_EMB = 64
_CHUNK = 27_776
_NCH = 36
_TAIL_BASE = _NCH * _CHUNK
_TAIL = _VOCAB - _TAIL_BASE
_K = 4
_L = 2


def _tc_body(in_hbm, out_hbm, buf, tail_buf, in_sems, out_sems, tail_sems):
    def in_copy(j):
        s = j % _K
        return pltpu.make_async_copy(
            in_hbm.at[:, pl.ds(j * _CHUNK, _CHUNK)], buf.at[s], in_sems.at[s]
        )

    def out_copy(j):
        s = j % _K
        return pltpu.make_async_copy(
            buf.at[s], out_hbm.at[:, pl.ds(j * _CHUNK, _CHUNK)], out_sems.at[s]
        )

    t_in = pltpu.make_async_copy(
        in_hbm.at[:, pl.ds(_TAIL_BASE, _TAIL)], tail_buf, tail_sems.at[0]
    )
    t_out = pltpu.make_async_copy(
        tail_buf, out_hbm.at[:, pl.ds(_TAIL_BASE, _TAIL)], tail_sems.at[1]
    )
    t_in.start()

    waited = set()
    for j in range(_L):
        in_copy(j).start()
    t_in.wait()
    t_out.start()
    for j in range(_NCH):
        in_copy(j).wait()
        out_copy(j).start()
        nxt = j + _L
        if nxt < _NCH:
            prev = nxt - _K
            if prev >= 0:
                out_copy(prev).wait()
                waited.add(prev)
            in_copy(nxt).start()
    for j in range(_NCH):
        if j not in waited:
            out_copy(j).wait()
    t_out.wait()


def kernel(lang, W_emb):
    del lang
    W_t = W_emb.T
    out = pl.pallas_call(
        _tc_body,
        in_specs=[pl.BlockSpec(memory_space=pltpu.MemorySpace.HBM)],
        out_specs=pl.BlockSpec(memory_space=pltpu.MemorySpace.HBM),
        out_shape=jax.ShapeDtypeStruct((_EMB, _VOCAB), jnp.float32),
        scratch_shapes=[
            pltpu.VMEM((_K, _EMB, _CHUNK), jnp.float32),
            pltpu.VMEM((_EMB, _TAIL), jnp.float32),
            pltpu.SemaphoreType.DMA((_K,)),
            pltpu.SemaphoreType.DMA((_K,)),
            pltpu.SemaphoreType.DMA((2,)),
        ],
    )(W_t)
    return out.T


# speedup vs baseline: 1.2525x; 1.0016x over previous
import jax
import jax.numpy as jnp
from jax.experimental import pallas as pl
from jax.experimental.pallas import tpu as pltpu

_VOCAB = 1_000_000
_EMB = 64
_CHUNK = 47_616
_NCH = 21
_TAIL_BASE = _NCH * _CHUNK
_TAIL = _VOCAB - _TAIL_BASE
_K = 4
_L = 2


def _tc_body(in_hbm, out_hbm, buf, tail_buf, in_sems, out_sems, tail_sems):
    def in_copy(j):
        s = j % _K
        return pltpu.make_async_copy(
            in_hbm.at[:, pl.ds(j * _CHUNK, _CHUNK)], buf.at[s], in_sems.at[s]
        )

    def out_copy(j):
        s = j % _K
        return pltpu.make_async_copy(
            buf.at[s], out_hbm.at[:, pl.ds(j * _CHUNK, _CHUNK)], out_sems.at[s]
        )

    t_in = pltpu.make_async_copy(
        in_hbm.at[:, pl.ds(_TAIL_BASE, _TAIL)], tail_buf, tail_sems.at[0]
    )
    t_out = pltpu.make_async_copy(
        tail_buf, out_hbm.at[:, pl.ds(_TAIL_BASE, _TAIL)], tail_sems.at[1]
    )
    t_in.start()

    waited = set()
    for j in range(_L):
        in_copy(j).start()
    t_in.wait()
    t_out.start()
    for j in range(_NCH):
        in_copy(j).wait()
        out_copy(j).start()
        nxt = j + _L
        if nxt < _NCH:
            prev = nxt - _K
            if prev >= 0:
                out_copy(prev).wait()
                waited.add(prev)
            in_copy(nxt).start()
    for j in range(_NCH):
        if j not in waited:
            out_copy(j).wait()
    t_out.wait()


def kernel(lang, W_emb):
    del lang
    W_t = W_emb.T
    out = pl.pallas_call(
        _tc_body,
        in_specs=[pl.BlockSpec(memory_space=pltpu.MemorySpace.HBM)],
        out_specs=pl.BlockSpec(memory_space=pltpu.MemorySpace.HBM),
        out_shape=jax.ShapeDtypeStruct((_EMB, _VOCAB), jnp.float32),
        scratch_shapes=[
            pltpu.VMEM((_K, _EMB, _CHUNK), jnp.float32),
            pltpu.VMEM((_EMB, _TAIL), jnp.float32),
            pltpu.SemaphoreType.DMA((_K,)),
            pltpu.SemaphoreType.DMA((_K,)),
            pltpu.SemaphoreType.DMA((2,)),
        ],
    )(W_t)
    return out.T
